# baseline (speedup 1.0000x reference)
import functools

import jax
import jax.numpy as jnp
import numpy as np
from jax.experimental import pallas as pl
from jax.experimental.pallas import tpu as pltpu
from jax.sharding import PartitionSpec as P

try:
    from jax import shard_map as _shard_map
except ImportError:
    from jax.experimental.shard_map import shard_map as _shard_map

_B, _N, _C = 8, 2048, 40
_K = 8


def _loss_kernel(nb, tgt_ref, a_ref, b_ref, pred_ref, out_ref):
    pcol = jax.lax.broadcasted_iota(jnp.int32, (1, _C), 1)
    total = jnp.float32(0.0)
    for b in range(nb):
        a = a_ref[b]
        bb = b_ref[b]
        an = jnp.sum(a * a, axis=1, keepdims=True)
        bn = jnp.sum(bb * bb, axis=0, keepdims=True)
        c2 = jax.lax.dot(-2.0 * a, bb,
                         preferred_element_type=jnp.float32)
        m1 = jnp.min(bn + c2, axis=1, keepdims=True)
        m2 = jnp.min(an + c2, axis=0, keepdims=True)
        s1 = jnp.sum(jnp.maximum(an + m1, 0.0))
        s2 = jnp.sum(jnp.maximum(bn + m2, 0.0))
        pv = jnp.sum(jnp.where(pcol == tgt_ref[b], pred_ref[b], 0.0))
        total += (s1 + s2) / (_N * _B) - pv / _B
    out_ref[...] = total.reshape(1, 1)


def _shard_call(nb, tgt, a, b, p):
    grid_spec = pltpu.PrefetchScalarGridSpec(
        num_scalar_prefetch=1,
        grid=(1,),
        in_specs=[
            pl.BlockSpec((nb, _N, _K), lambda i, t: (0, 0, 0)),
            pl.BlockSpec((nb, _K, _N), lambda i, t: (0, 0, 0)),
            pl.BlockSpec((nb, 1, _C), lambda i, t: (0, 0, 0)),
        ],
        out_specs=pl.BlockSpec((1, 1), lambda i, t: (0, 0)),
    )
    return pl.pallas_call(
        functools.partial(_loss_kernel, nb),
        grid_spec=grid_spec,
        out_shape=jax.ShapeDtypeStruct((1, 1), jnp.float32),
        compiler_params=pltpu.CompilerParams(
            dimension_semantics=("arbitrary",),
        ),
    )(tgt, a, b, p)


def kernel(reg, point1, pred, target):
    a3 = jnp.pad(reg, ((0, 0), (0, 0), (0, _K - 3)))
    b3 = jnp.pad(point1, ((0, 0), (0, 0), (0, _K - 3))).transpose(0, 2, 1)
    pred3 = pred.reshape(_B, 1, _C)

    ndev = 2 if jax.device_count() >= 2 else 1
    mesh = jax.sharding.Mesh(np.array(jax.devices()[:ndev]), ("d",))
    fn = _shard_map(
        functools.partial(_shard_call, _B // ndev),
        mesh=mesh,
        in_specs=(P("d"), P("d"), P("d"), P("d")),
        out_specs=P("d"),
        check_vma=False,
    )
    out = fn(target, a3, b3, pred3)
    return jnp.sum(out)


# speedup vs baseline: 9.7201x; 9.7201x over previous
import functools

import jax
import jax.numpy as jnp
from jax import lax
from jax.experimental import pallas as pl
from jax.experimental.pallas import tpu as pltpu
from jax.experimental.pallas import tpu_sc as plsc

_B, _N, _C = 8, 2048, 40
_K = 8
_L = 16


def _chamfer_kernel(a_ref, b_ref, out_ref):
    total = jnp.float32(0.0)
    for b in range(_B):
        a = a_ref[b]
        bb = b_ref[b]
        an = jnp.sum(a * a, axis=1, keepdims=True)
        bn = jnp.sum(bb * bb, axis=0, keepdims=True)
        c2 = jax.lax.dot(-2.0 * a, bb,
                         preferred_element_type=jnp.float32)
        m1 = jnp.min(bn + c2, axis=1, keepdims=True)
        m2 = jnp.min(an + c2, axis=0, keepdims=True)
        s1 = jnp.sum(jnp.maximum(an + m1, 0.0))
        s2 = jnp.sum(jnp.maximum(bn + m2, 0.0))
        total += (s1 + s2) / (_N * _B)
    out_ref[...] = total.reshape(1, 1)


@functools.partial(
    pl.kernel,
    mesh=plsc.VectorSubcoreMesh(core_axis_name="c", subcore_axis_name="s"),
    out_type=jax.ShapeDtypeStruct((_L,), jnp.float32),
    scratch_types=[
        pltpu.VMEM((_C, _L), jnp.float32),
        pltpu.VMEM((_L,), jnp.int32),
        pltpu.VMEM((_L,), jnp.float32),
    ],
)
def _nll_gather_sc(pred_hbm, idx_hbm, out_hbm, predv, idxv, outv):
    @pl.when((lax.axis_index("c") == 0) & (lax.axis_index("s") == 0))
    def _():
        pltpu.sync_copy(pred_hbm, predv)
        pltpu.sync_copy(idx_hbm, idxv)
        idx = idxv[...]
        vals = jnp.zeros((_L,), jnp.float32)
        for c in range(_C):
            vals = jnp.where(idx == c, predv[c], vals)
        outv[...] = vals
        pltpu.sync_copy(outv, out_hbm)


def kernel(reg, point1, pred, target):
    a3 = jnp.pad(reg, ((0, 0), (0, 0), (0, _K - 3)))
    b3 = jnp.pad(point1, ((0, 0), (0, 0), (0, _K - 3))).transpose(0, 2, 1)
    pred_cm = jnp.pad(pred.T, ((0, 0), (0, _L - _B)))
    tgt_pad = jnp.pad(target, (0, _L - _B), constant_values=_C)

    gathered = _nll_gather_sc(pred_cm, tgt_pad)

    chamfer = pl.pallas_call(
        _chamfer_kernel,
        grid=(1,),
        in_specs=[
            pl.BlockSpec((_B, _N, _K), lambda i: (0, 0, 0)),
            pl.BlockSpec((_B, _K, _N), lambda i: (0, 0, 0)),
        ],
        out_specs=pl.BlockSpec((1, 1), lambda i: (0, 0)),
        out_shape=jax.ShapeDtypeStruct((1, 1), jnp.float32),
        compiler_params=pltpu.CompilerParams(
            dimension_semantics=("arbitrary",),
        ),
    )(a3, b3)
    return chamfer[0, 0] - jnp.sum(gathered) / _B


# speedup vs baseline: 14.3174x; 1.4730x over previous
import jax
import jax.numpy as jnp
from jax.experimental import pallas as pl
from jax.experimental.pallas import tpu as pltpu

_B, _N, _C = 8, 2048, 40
_K = 8


def _loss_kernel(tgt_ref, a_ref, b_ref, pred_ref, out_ref):
    pcol = jax.lax.broadcasted_iota(jnp.int32, (1, _C), 1)
    total = jnp.float32(0.0)
    for b in range(_B):
        a = a_ref[b]
        bb = b_ref[b]
        an = jnp.sum(a * a, axis=1, keepdims=True)
        bn = jnp.sum(bb * bb, axis=0, keepdims=True)
        c2 = jax.lax.dot(-2.0 * a, bb,
                         preferred_element_type=jnp.float32)
        m1 = jnp.min(bn + c2, axis=1, keepdims=True)
        m2 = jnp.min(an + c2, axis=0, keepdims=True)
        s1 = jnp.sum(jnp.maximum(an + m1, 0.0))
        s2 = jnp.sum(jnp.maximum(bn + m2, 0.0))
        pv = jnp.sum(jnp.where(pcol == tgt_ref[b], pred_ref[b], 0.0))
        total += (s1 + s2) / (_N * _B) - pv / _B
    out_ref[...] = total.reshape(1, 1)


def kernel(reg, point1, pred, target):
    a3 = jnp.pad(reg, ((0, 0), (0, 0), (0, _K - 3)))
    b3 = jnp.pad(point1, ((0, 0), (0, 0), (0, _K - 3))).transpose(0, 2, 1)
    pred3 = pred.reshape(_B, 1, _C)

    grid_spec = pltpu.PrefetchScalarGridSpec(
        num_scalar_prefetch=1,
        grid=(1,),
        in_specs=[
            pl.BlockSpec((_B, _N, _K), lambda i, tgt: (0, 0, 0)),
            pl.BlockSpec((_B, _K, _N), lambda i, tgt: (0, 0, 0)),
            pl.BlockSpec((_B, 1, _C), lambda i, tgt: (0, 0, 0)),
        ],
        out_specs=pl.BlockSpec((1, 1), lambda i, tgt: (0, 0)),
    )
    out = pl.pallas_call(
        _loss_kernel,
        grid_spec=grid_spec,
        out_shape=jax.ShapeDtypeStruct((1, 1), jnp.float32),
        compiler_params=pltpu.CompilerParams(
            dimension_semantics=("arbitrary",),
        ),
    )(target, a3, b3, pred3)
    return out[0, 0]


# speedup vs baseline: 15.7318x; 1.0988x over previous
import jax
import jax.numpy as jnp
from jax.experimental import pallas as pl
from jax.experimental.pallas import tpu as pltpu

_B, _N, _C = 8, 2048, 40
_K = 8


def _loss_kernel(tgt_ref, a_ref, b_ref, pred_ref, out_ref):
    pcol = jax.lax.broadcasted_iota(jnp.int32, (1, _C), 1)
    col = jax.lax.broadcasted_iota(jnp.int32, (_N, _K), 1)
    row = jax.lax.broadcasted_iota(jnp.int32, (_K, _N), 0)
    total = jnp.float32(0.0)
    for b in range(_B):
        a = a_ref[b]
        bb = b_ref[b]
        an = jnp.sum(a * a, axis=1, keepdims=True)
        bn = jnp.sum(bb * bb, axis=0, keepdims=True)
        an_hi = an.astype(jnp.bfloat16).astype(jnp.float32)
        an_lo = an - an_hi
        bn_hi = bn.astype(jnp.bfloat16).astype(jnp.float32)
        bn_lo = bn - bn_hi
        lhs = (-2.0 * a
               + jnp.where(col == 3, an_hi, 0.0)
               + jnp.where(col == 4, an_lo, 0.0)
               + jnp.where((col == 5) | (col == 6), 1.0, 0.0))
        rhs = (bb
               + jnp.where((row == 3) | (row == 4), 1.0, 0.0)
               + jnp.where(row == 5, bn_hi, 0.0)
               + jnp.where(row == 6, bn_lo, 0.0))
        g = jax.lax.dot(lhs, rhs, preferred_element_type=jnp.float32)
        m1 = jnp.min(g, axis=1, keepdims=True)
        m2 = jnp.min(g, axis=0, keepdims=True)
        s1 = jnp.sum(jnp.maximum(m1, 0.0))
        s2 = jnp.sum(jnp.maximum(m2, 0.0))
        pv = jnp.sum(jnp.where(pcol == tgt_ref[b], pred_ref[b], 0.0))
        total += (s1 + s2) / (_N * _B) - pv / _B
    out_ref[...] = total.reshape(1, 1)


def kernel(reg, point1, pred, target):
    a3 = jnp.pad(reg, ((0, 0), (0, 0), (0, _K - 3)))
    b3 = jnp.pad(point1, ((0, 0), (0, 0), (0, _K - 3))).transpose(0, 2, 1)
    pred3 = pred.reshape(_B, 1, _C)

    grid_spec = pltpu.PrefetchScalarGridSpec(
        num_scalar_prefetch=1,
        grid=(1,),
        in_specs=[
            pl.BlockSpec((_B, _N, _K), lambda i, tgt: (0, 0, 0)),
            pl.BlockSpec((_B, _K, _N), lambda i, tgt: (0, 0, 0)),
            pl.BlockSpec((_B, 1, _C), lambda i, tgt: (0, 0, 0)),
        ],
        out_specs=pl.BlockSpec((1, 1), lambda i, tgt: (0, 0)),
    )
    out = pl.pallas_call(
        _loss_kernel,
        grid_spec=grid_spec,
        out_shape=jax.ShapeDtypeStruct((1, 1), jnp.float32),
        compiler_params=pltpu.CompilerParams(
            dimension_semantics=("arbitrary",),
        ),
    )(target, a3, b3, pred3)
    return out[0, 0]
